# baseline (speedup 1.0000x reference)
import functools

import jax
import jax.numpy as jnp
from jax import lax
from jax.experimental import pallas as pl
from jax.experimental.pallas import tpu as pltpu
from jax.experimental.pallas import tpu_sc as plsc

IDX_MINOR = 128


@functools.cache
def _make_sc_lookup(N: int, L: int, V: int, D: int):
    info = plsc.get_sparse_core_info()
    NC = info.num_cores
    num_workers = NC * info.num_subcores
    B = N * L
    DT = D // 8
    NT = N // IDX_MINOR
    NTG = 2
    n_tasks = L * (NT // NTG)
    tasks_per_w = n_tasks // num_workers
    rows_per_task = NTG * IDX_MINOR
    idxrows_per_w = tasks_per_w * NTG

    mesh = plsc.VectorSubcoreMesh(core_axis_name="c", subcore_axis_name="s")

    @functools.partial(
        pl.kernel,
        out_type=jax.ShapeDtypeStruct((L, DT, NT, 8, IDX_MINOR), jnp.float32),
        mesh=mesh,
        scratch_types=[
            pltpu.VMEM((idxrows_per_w, IDX_MINOR), jnp.int32),
            pltpu.VMEM((4, rows_per_task, D), jnp.float32),
            pltpu.VMEM((2, DT, NTG, 8, IDX_MINOR), jnp.float32),
            pltpu.SemaphoreType.DMA,
            pltpu.SemaphoreType.DMA,
        ],
        compiler_params=pltpu.CompilerParams(
            use_tc_tiling_on_sc=False, needs_layout_passes=False
        ),
    )
    def lookup_kernel(idx_hbm, table_hbm, out_hbm, idx_v, rows_v, tr_v, gsem, wsem):
        wid = lax.axis_index("s") * NC + lax.axis_index("c")
        pltpu.sync_copy(
            idx_hbm.at[pl.ds(wid * idxrows_per_w, idxrows_per_w)], idx_v
        )
        task0 = wid * tasks_per_w

        def fire_gathers(tl, buf):
            for j in range(NTG):
                pltpu.async_copy(
                    table_hbm.at[idx_v.at[tl * NTG + j]],
                    rows_v.at[buf].at[pl.ds(j * IDX_MINOR, IDX_MINOR)],
                    gsem,
                )

        for p in range(3):
            fire_gathers(p, p)

        @pl.loop(0, tasks_per_w, step=4)
        def _quad(c0):
            for b in range(4):
                t = c0 + b
                tb = b % 2
                pltpu.make_async_copy(
                    table_hbm.at[pl.ds(0, rows_per_task)], rows_v.at[b], gsem
                ).wait()

                @pl.when(t + 3 < tasks_per_w)
                def _():
                    fire_gathers(t + 3, (b + 3) % 4)

                @pl.when(t >= 2)
                def _():
                    pltpu.make_async_copy(
                        out_hbm.at[0, :, pl.ds(0, NTG)], tr_v.at[tb], wsem
                    ).wait()

                @plsc.parallel_loop(0, rows_per_task // 16, unroll=2)
                def _jk(jk):
                    j = jk // (IDX_MINOR // 16)
                    k = jk % (IDX_MINOR // 16)
                    n_idx = lax.iota(jnp.int32, 16) + jk * 16
                    for g in range(2):
                        vals = []
                        for dd in range(D // 2):
                            d = g * (D // 2) + dd
                            d_idx = jnp.full((16,), d, jnp.int32)
                            vals.append(
                                plsc.load_gather(rows_v.at[b], [n_idx, d_idx])
                            )
                        for dd in range(D // 2):
                            d = g * (D // 2) + dd
                            tr_v[tb, d // 8, j, d % 8, pl.ds(k * 16, 16)] = vals[dd]

                task = task0 + t
                l = task // (NT // NTG)
                ntg = task % (NT // NTG)
                pltpu.async_copy(
                    tr_v.at[tb], out_hbm.at[l, :, pl.ds(ntg * NTG, NTG)], wsem
                )

        for b in range(2):
            pltpu.make_async_copy(
                out_hbm.at[0, :, pl.ds(0, NTG)], tr_v.at[b], wsem
            ).wait()

    return lookup_kernel


@jax.jit
def kernel(x, embeddings):
    N_, L_ = x.shape
    V, D = embeddings.shape
    B = N_ * L_
    idx = x.T.reshape(B // IDX_MINOR, IDX_MINOR).astype(jnp.int32)
    out5d = _make_sc_lookup(N_, L_, V, D)(idx, embeddings)
    return out5d.transpose(2, 4, 0, 1, 3).reshape(N_, L_, D)


# speedup vs baseline: 1.0162x; 1.0162x over previous
import functools

import jax
import jax.numpy as jnp
from jax import lax
from jax.experimental import pallas as pl
from jax.experimental.pallas import tpu as pltpu
from jax.experimental.pallas import tpu_sc as plsc

IDX_MINOR = 128


@functools.cache
def _make_sc_lookup(N: int, L: int, V: int, D: int):
    info = plsc.get_sparse_core_info()
    NC = info.num_cores
    num_workers = NC * info.num_subcores
    B = N * L
    DT = D // 8
    NT = N // IDX_MINOR
    NTG = 2
    n_tasks = L * (NT // NTG)
    tasks_per_w = n_tasks // num_workers
    rows_per_task = NTG * IDX_MINOR
    idxrows_per_w = tasks_per_w * NTG

    mesh = plsc.VectorSubcoreMesh(core_axis_name="c", subcore_axis_name="s")

    @functools.partial(
        pl.kernel,
        out_type=jax.ShapeDtypeStruct((L, DT, NT, 8, IDX_MINOR), jnp.float32),
        mesh=mesh,
        scratch_types=[
            pltpu.VMEM((idxrows_per_w, IDX_MINOR), jnp.int32),
            pltpu.VMEM((4, rows_per_task, D), jnp.float32),
            pltpu.VMEM((2, DT, NTG, 8, IDX_MINOR), jnp.float32),
            pltpu.SemaphoreType.DMA,
            pltpu.SemaphoreType.DMA,
        ],
        compiler_params=pltpu.CompilerParams(
            use_tc_tiling_on_sc=False, needs_layout_passes=False
        ),
    )
    def lookup_kernel(idx_hbm, table_hbm, out_hbm, idx_v, rows_v, tr_v, gsem, wsem):
        wid = lax.axis_index("s") * NC + lax.axis_index("c")
        pltpu.sync_copy(
            idx_hbm.at[pl.ds(wid * idxrows_per_w, idxrows_per_w)], idx_v
        )
        task0 = wid * tasks_per_w

        def fire_gathers(tl, buf):
            for j in range(NTG):
                pltpu.async_copy(
                    table_hbm.at[idx_v.at[tl * NTG + j]],
                    rows_v.at[buf].at[pl.ds(j * IDX_MINOR, IDX_MINOR)],
                    gsem,
                )

        for p in range(3):
            fire_gathers(p, p)

        @pl.loop(0, tasks_per_w, step=4)
        def _quad(c0):
            for b in range(4):
                t = c0 + b
                tb = b % 2
                pltpu.make_async_copy(
                    table_hbm.at[pl.ds(0, rows_per_task)], rows_v.at[b], gsem
                ).wait()

                @pl.when(t + 3 < tasks_per_w)
                def _():
                    fire_gathers(t + 3, (b + 3) % 4)

                @pl.when(t >= 2)
                def _():
                    pltpu.make_async_copy(
                        out_hbm.at[0, :, pl.ds(0, NTG)], tr_v.at[tb], wsem
                    ).wait()

                @plsc.parallel_loop(0, rows_per_task // 16, unroll=1)
                def _jk(jk):
                    j = jk // (IDX_MINOR // 16)
                    k = jk % (IDX_MINOR // 16)
                    n_idx = lax.iota(jnp.int32, 16) + jk * 16
                    vals = []
                    for d in range(D):
                        d_idx = jnp.full((16,), d, jnp.int32)
                        vals.append(plsc.load_gather(rows_v.at[b], [n_idx, d_idx]))
                    for d in range(D):
                        tr_v[tb, d // 8, j, d % 8, pl.ds(k * 16, 16)] = vals[d]

                task = task0 + t
                l = task // (NT // NTG)
                ntg = task % (NT // NTG)
                pltpu.async_copy(
                    tr_v.at[tb], out_hbm.at[l, :, pl.ds(ntg * NTG, NTG)], wsem
                )

        for b in range(2):
            pltpu.make_async_copy(
                out_hbm.at[0, :, pl.ds(0, NTG)], tr_v.at[b], wsem
            ).wait()

    return lookup_kernel


@jax.jit
def kernel(x, embeddings):
    N_, L_ = x.shape
    V, D = embeddings.shape
    B = N_ * L_
    idx = x.T.reshape(B // IDX_MINOR, IDX_MINOR).astype(jnp.int32)
    out5d = _make_sc_lookup(N_, L_, V, D)(idx, embeddings)
    return out5d.transpose(2, 4, 0, 1, 3).reshape(N_, L_, D)
